# baseline (speedup 1.0000x reference)
import functools

import jax
import jax.numpy as jnp
from jax import lax
from jax.experimental import pallas as pl
from jax.experimental.pallas import tpu as pltpu
from jax.experimental.pallas import tpu_sc as plsc

N, E, D, H, O, R = 10000, 320000, 128, 128, 128, 2

NC, NS, LANES = 2, 16, 16
CH = 128
DA = D + LANES
TRASH = N
N_PAD = 10112
E_PAD = 327680
CHUNKS_A = E_PAD // (NS * CH)
CHUNKS_B = E_PAD // (NC * NS * CH)
PAIRS_A = CHUNKS_A // 2
PAIRS_B = CHUNKS_B // 2
ROWS_A = E_PAD // CH
NCH = N_PAD // CH
KMAX = (NCH + NS - 1) // NS

_mesh = plsc.VectorSubcoreMesh(core_axis_name="c", subcore_axis_name="s")
_sc_params = pltpu.CompilerParams(use_tc_tiling_on_sc=False)


@functools.partial(
    pl.kernel,
    out_type=jax.ShapeDtypeStruct((NC, N_PAD, DA), jnp.float32),
    mesh=_mesh,
    compiler_params=_sc_params,
    scratch_types=[
        pltpu.VMEM((2, CH, DA), jnp.float32),
        pltpu.VMEM((2, CH), jnp.int32),
        pltpu.VMEM((2, CH), jnp.int32),
        pltpu.VMEM_SHARED((N_PAD, DA), jnp.float32),
        pltpu.SemaphoreType.DMA,
        pltpu.SemaphoreType.DMA,
        pltpu.SemaphoreType.DMA,
        pltpu.SemaphoreType.DMA,
    ],
)
def _sc_pass_a(xa_hbm, idxt_hbm, za_hbm,
               agg_out, rows_v, idx0_v, idx1_v, agg_sh,
               isem, gsem, s0sem, s1sem):
    c = lax.axis_index("c")
    s = lax.axis_index("s")

    pltpu.sync_copy(za_hbm, rows_v.at[0])
    for k in range(KMAX):
        ch = s + NS * k

        @pl.when(ch < NCH)
        def _():
            pltpu.sync_copy(rows_v.at[0], agg_sh.at[pl.ds(ch * CH, CH)])

    plsc.subcore_barrier()

    def body(g, carry):
        row = s * CHUNKS_A + 2 * g

        @pl.when(g > 0)
        def _():
            pltpu.make_async_copy(za_hbm, rows_v.at[0], s0sem).wait()

        di0 = pltpu.async_copy(idxt_hbm.at[c, row], idx0_v, isem)

        @pl.when(g > 0)
        def _():
            pltpu.make_async_copy(za_hbm, rows_v.at[1], s1sem).wait()

        di1 = pltpu.async_copy(idxt_hbm.at[c, row + 1], idx1_v, isem)
        di0.wait()
        g0 = pltpu.async_copy(xa_hbm.at[idx0_v.at[0]], rows_v.at[0], gsem)
        di1.wait()
        g1 = pltpu.async_copy(xa_hbm.at[idx1_v.at[0]], rows_v.at[1], gsem)
        g0.wait()
        pltpu.async_copy(rows_v.at[0], agg_sh.at[idx0_v.at[1]], s0sem,
                         add=True)
        g1.wait()
        pltpu.async_copy(rows_v.at[1], agg_sh.at[idx1_v.at[1]], s1sem,
                         add=True)
        return carry

    lax.fori_loop(0, PAIRS_A, body, 0)
    pltpu.make_async_copy(za_hbm, rows_v.at[0], s0sem).wait()
    pltpu.make_async_copy(za_hbm, rows_v.at[1], s1sem).wait()
    plsc.subcore_barrier()

    for k in range(KMAX):
        ch = s + NS * k

        @pl.when(ch < NCH)
        def _():
            r = ch * CH
            pltpu.sync_copy(agg_sh.at[pl.ds(r, CH)], rows_v.at[0])
            pltpu.sync_copy(rows_v.at[0], agg_out.at[c, pl.ds(r, CH)])


@functools.partial(
    pl.kernel,
    out_type=jax.ShapeDtypeStruct((NC, N_PAD, D), jnp.float32),
    mesh=_mesh,
    compiler_params=_sc_params,
    scratch_types=[
        pltpu.VMEM((2, CH, D), jnp.float32),
        pltpu.VMEM((2, CH), jnp.int32),
        pltpu.VMEM((2, CH), jnp.int32),
        pltpu.VMEM_SHARED((N_PAD, D), jnp.float32),
        pltpu.SemaphoreType.DMA,
        pltpu.SemaphoreType.DMA,
        pltpu.SemaphoreType.DMA,
        pltpu.SemaphoreType.DMA,
    ],
)
def _sc_pass_b(h_hbm, idxt_hbm, z_hbm,
               part_out, rows_v, idx0_v, idx1_v, agg_sh,
               isem, gsem, s0sem, s1sem):
    c = lax.axis_index("c")
    s = lax.axis_index("s")

    pltpu.sync_copy(z_hbm, rows_v.at[0])
    for k in range(KMAX):
        ch = s + NS * k

        @pl.when(ch < NCH)
        def _():
            pltpu.sync_copy(rows_v.at[0], agg_sh.at[pl.ds(ch * CH, CH)])

    plsc.subcore_barrier()

    def body(g, carry):
        row = (c * NS + s) * CHUNKS_B + 2 * g

        @pl.when(g > 0)
        def _():
            pltpu.make_async_copy(z_hbm, rows_v.at[0], s0sem).wait()

        di0 = pltpu.async_copy(idxt_hbm.at[row], idx0_v, isem)

        @pl.when(g > 0)
        def _():
            pltpu.make_async_copy(z_hbm, rows_v.at[1], s1sem).wait()

        di1 = pltpu.async_copy(idxt_hbm.at[row + 1], idx1_v, isem)
        di0.wait()
        g0 = pltpu.async_copy(h_hbm.at[idx0_v.at[0]], rows_v.at[0], gsem)
        di1.wait()
        g1 = pltpu.async_copy(h_hbm.at[idx1_v.at[0]], rows_v.at[1], gsem)
        g0.wait()
        pltpu.async_copy(rows_v.at[0], agg_sh.at[idx0_v.at[1]], s0sem,
                         add=True)
        g1.wait()
        pltpu.async_copy(rows_v.at[1], agg_sh.at[idx1_v.at[1]], s1sem,
                         add=True)
        return carry

    lax.fori_loop(0, PAIRS_B, body, 0)
    pltpu.make_async_copy(z_hbm, rows_v.at[0], s0sem).wait()
    pltpu.make_async_copy(z_hbm, rows_v.at[1], s1sem).wait()
    plsc.subcore_barrier()

    for k in range(KMAX):
        ch = s + NS * k

        @pl.when(ch < NCH)
        def _():
            r = ch * CH
            pltpu.sync_copy(agg_sh.at[pl.ds(r, CH)], rows_v.at[0])
            pltpu.sync_copy(rows_v.at[0], part_out.at[c, pl.ds(r, CH)])


_BLK = 400


def _tc1_body(x_ref, a0_ref, a1_ref, c0_ref, c1_ref,
              wr_ref, w0_ref, w1_ref, b_ref, h_ref):
    cnt0 = jnp.maximum(c0_ref[:, 0:1], 1.0)
    cnt1 = jnp.maximum(c1_ref[:, 0:1], 1.0)
    acc = jnp.dot(x_ref[...], wr_ref[...], preferred_element_type=jnp.float32)
    acc += jnp.dot(a0_ref[...] / cnt0, w0_ref[...],
                   preferred_element_type=jnp.float32)
    acc += jnp.dot(a1_ref[...] / cnt1, w1_ref[...],
                   preferred_element_type=jnp.float32)
    h_ref[...] = jnp.maximum(acc + b_ref[...], 0.0)


def _tc2_body(p0_ref, p1_ref, h_ref, wrel_ref, wroot_ref, b_ref, o_ref):
    acc = jnp.dot(p0_ref[...] + p1_ref[...], wrel_ref[...],
                  preferred_element_type=jnp.float32)
    acc += jnp.dot(h_ref[...], wroot_ref[...],
                   preferred_element_type=jnp.float32)
    o_ref[...] = acc + b_ref[...]


def _row_blk(i):
    return (i, 0)


def _whole(i):
    return (0, 0)


def kernel(x, relationsedge_indices_relations, edge_type, W_rel1, W_root1, b1,
           W_rel2, W_root2, b2):
    edge_index = relationsedge_indices_relations[-1]
    src = edge_index[0].astype(jnp.int32)
    dst = edge_index[1].astype(jnp.int32)
    et = edge_type.astype(jnp.int32)

    pad = E_PAD - E
    src_p = jnp.concatenate([src, jnp.zeros((pad,), jnp.int32)])
    dst_p = jnp.concatenate([dst, jnp.full((pad,), TRASH, jnp.int32)])
    et_p = jnp.concatenate([et, jnp.full((pad,), R, jnp.int32)])
    src2 = src_p.reshape(-1, CH)

    idxt_a = jnp.stack([
        jnp.stack([src2,
                   jnp.where(et_p == r, dst_p, TRASH).reshape(-1, CH)],
                  axis=1)
        for r in range(R)
    ])
    idxt_b = jnp.stack([src2, dst_p.reshape(-1, CH)], axis=1)

    xa = jnp.concatenate([x, jnp.ones((N, LANES), jnp.float32)], axis=1)
    za = jnp.zeros((CH, DA), jnp.float32)
    z = jnp.zeros((CH, D), jnp.float32)

    agg = _sc_pass_a(xa, idxt_a, za)

    rowspec = pl.BlockSpec((_BLK, D), _row_blk)
    cntspec = pl.BlockSpec((_BLK, LANES), _row_blk)
    wspec = pl.BlockSpec((D, D), _whole)
    bspec = pl.BlockSpec((1, D), _whole)

    h = pl.pallas_call(
        _tc1_body,
        grid=(N // _BLK,),
        in_specs=[rowspec, rowspec, rowspec, cntspec, cntspec,
                  wspec, wspec, wspec, bspec],
        out_specs=rowspec,
        out_shape=jax.ShapeDtypeStruct((N, D), jnp.float32),
    )(x, agg[0, :N, :D], agg[1, :N, :D], agg[0, :N, D:], agg[1, :N, D:],
      W_root1, W_rel1[0], W_rel1[1], b1.reshape(1, D))

    part = _sc_pass_b(h, idxt_b, z)

    out2 = pl.pallas_call(
        _tc2_body,
        grid=(N // _BLK,),
        in_specs=[rowspec, rowspec, rowspec, wspec, wspec, bspec],
        out_specs=rowspec,
        out_shape=jax.ShapeDtypeStruct((N, O), jnp.float32),
    )(part[0, :N], part[1, :N], h, W_rel2, W_root2, b2.reshape(1, O))

    return out2.reshape(-1, 1, O)
